# baseline (speedup 1.0000x reference)
import functools

import jax
import jax.numpy as jnp
from jax import lax
from jax.experimental import pallas as pl
from jax.experimental.pallas import tpu as pltpu
from jax.experimental.pallas import tpu_sc as plsc

TREE_DEPTH = 4
D_MODEL = 1024
NUM_NODES = (1 << TREE_DEPTH) - 1
TAU_MEM = 0.99
TAU_ADAPT = 0.95
RMS_EPS = 1.1920929e-07

SS1 = 256
SS2 = 128
NIN = 8


def _leaf_sc_kernel(x_hbm, out_hbm, chunk_v, acc_v):
    c = lax.axis_index("c")
    s = lax.axis_index("s")
    wid = s * 2 + c
    b = wid // 8
    leaf = wid % 8
    base = leaf * 256
    accs = tuple(jnp.zeros((16,), jnp.float32) for _ in range(64))
    for ch in range(8):
        pltpu.sync_copy(x_hbm.at[b, pl.ds(base + ch * 32, 32), :], chunk_v)

        def body(t, a):
            return tuple(a[d] + chunk_v[t, pl.ds(d * 16, 16)]
                         for d in range(64))

        accs = lax.fori_loop(0, 32, body, accs)
    for d in range(64):
        acc_v[pl.ds(d * 16, 16)] = accs[d] * (1.0 / 256.0)
    pltpu.sync_copy(acc_v, out_hbm.at[b, leaf])


def _leaf_sc(x):
    mesh = plsc.VectorSubcoreMesh(core_axis_name="c", subcore_axis_name="s")
    return pl.kernel(
        _leaf_sc_kernel,
        out_type=jax.ShapeDtypeStruct((4, 8, D_MODEL), jnp.float32),
        mesh=mesh,
        scratch_types=[
            pltpu.VMEM((32, D_MODEL), jnp.float32),
            pltpu.VMEM((D_MODEL,), jnp.float32),
        ],
    )(x)


def _fused_kernel(x_hbm, leaves_ref, w_ref, nw_ref, bt_ref, as_ref, rw_ref,
                  out_hbm, xfirst, ring, sem_in1, sem_in2, sem_out1,
                  sem_out2):
    B = 4
    D = D_MODEL
    HALF = 8 * SS1

    def in1_copy(j):
        return pltpu.make_async_copy(
            x_hbm.at[:, pl.ds(j * SS1, SS1), :],
            xfirst.at[:, pl.ds(j * SS1, SS1), :],
            sem_in1.at[j])

    def in2_copy(k):
        return pltpu.make_async_copy(
            x_hbm.at[:, pl.ds(HALF + k * SS2, SS2), :],
            ring.at[k % NIN],
            sem_in2.at[k % NIN])

    def out1_copy(j):
        return pltpu.make_async_copy(
            xfirst.at[:, pl.ds(j * SS1, SS1), :],
            out_hbm.at[:, pl.ds(j * SS1, SS1), :],
            sem_out1.at[j])

    def out2_copy(k):
        return pltpu.make_async_copy(
            ring.at[k % NIN],
            out_hbm.at[:, pl.ds(HALF + k * SS2, SS2), :],
            sem_out2.at[k % NIN])

    for j in range(8):
        in1_copy(j).start()
    for k in range(NIN):
        in2_copy(k).start()

    leaves = leaves_ref[...]
    states = [None] * NUM_NODES
    for n in range(7, 15):
        states[n] = leaves[:, n - 7, :]
    w = w_ref[...]
    for level in range(2, -1, -1):
        lo = (1 << level) - 1
        hi = (1 << (level + 1)) - 1
        nodes = list(range(lo, hi))
        nlev = len(nodes)
        l_st = jnp.stack([states[2 * n + 1] for n in nodes], axis=1)
        r_st = jnp.stack([states[2 * n + 2] for n in nodes], axis=1)
        fused = jnp.concatenate([l_st, r_st], axis=-1)
        fused2 = fused.reshape(B * nlev, 2 * D)
        proj = jax.lax.dot_general(
            fused2, w, (((1,), (1,)), ((), ())),
            preferred_element_type=jnp.float32,
        ).reshape(B, nlev, D)
        bt = bt_ref[level, :]
        asw = as_ref[level, :]
        v = jnp.zeros((B, D), jnp.float32)
        a = jnp.zeros((B, D), jnp.float32)
        for t in range(nlev):
            v = TAU_MEM * v + proj[:, t, :]
            thresh = bt + asw * a
            s = (v - thresh > 0).astype(jnp.float32)
            v = v * (1.0 - s)
            a = TAU_ADAPT * a + s
            states[nodes[t]] = proj[:, t, :] * s
    nw = nw_ref[...]
    mx = jnp.max(nw, axis=0, keepdims=True)
    e = jnp.exp(nw - mx)
    wts = e / jnp.sum(e, axis=0, keepdims=True)
    mix = jnp.zeros((B, D), jnp.float32)
    for n in range(NUM_NODES):
        mix = mix + wts[n, :][None, :] * states[n]
    rw = rw_ref[...]

    def rms(y):
        ms = jnp.mean(y * y, axis=-1, keepdims=True)
        return y * jax.lax.rsqrt(ms + RMS_EPS) * rw[None, None, :]

    for j in range(8):
        in1_copy(j).wait()
        xblk = xfirst[:, pl.ds(j * SS1, SS1), :]
        xfirst[:, pl.ds(j * SS1, SS1), :] = rms(xblk + mix[:, None, :])
        out1_copy(j).start()

    for k in range(16):
        in2_copy(k).wait()
        blk = ring[k % NIN]
        ring[k % NIN] = rms(blk + mix[:, None, :])
        out2_copy(k).start()
        if k >= 1 and (k - 1) + NIN < 16:
            out2_copy(k - 1).wait()
            in2_copy(k - 1 + NIN).start()
    for k in range(NIN, 16):
        out2_copy(k).wait()
    for j in range(8):
        out1_copy(j).wait()


def kernel(x, W_proj, node_weights, rms_weight, base_thresh, adapt_strength):
    B, T, D = x.shape
    assert T == 4096 and D == D_MODEL, "kernel assumes T=4096, D=1024"

    leaves = _leaf_sc(x)

    out = pl.pallas_call(
        _fused_kernel,
        in_specs=[
            pl.BlockSpec(memory_space=pl.ANY),
            pl.BlockSpec(memory_space=pltpu.VMEM),
            pl.BlockSpec(memory_space=pltpu.VMEM),
            pl.BlockSpec(memory_space=pltpu.VMEM),
            pl.BlockSpec(memory_space=pltpu.VMEM),
            pl.BlockSpec(memory_space=pltpu.VMEM),
            pl.BlockSpec(memory_space=pltpu.VMEM),
        ],
        out_specs=pl.BlockSpec(memory_space=pl.ANY),
        out_shape=jax.ShapeDtypeStruct((B, T, D), jnp.float32),
        scratch_shapes=[
            pltpu.VMEM((B, 8 * SS1, D), jnp.float32),
            pltpu.VMEM((NIN, B, SS2, D), jnp.float32),
            pltpu.SemaphoreType.DMA((8,)),
            pltpu.SemaphoreType.DMA((NIN,)),
            pltpu.SemaphoreType.DMA((8,)),
            pltpu.SemaphoreType.DMA((NIN,)),
        ],
        compiler_params=pltpu.CompilerParams(vmem_limit_bytes=62 * 1024 * 1024),
    )(x, leaves, W_proj, node_weights, base_thresh, adapt_strength,
      rms_weight)
    return out


# speedup vs baseline: 1.9961x; 1.9961x over previous
import jax
import jax.numpy as jnp
from jax.experimental import pallas as pl
from jax.experimental.pallas import tpu as pltpu

TREE_DEPTH = 4
D_MODEL = 1024
NUM_NODES = (1 << TREE_DEPTH) - 1
TAU_MEM = 0.99
TAU_ADAPT = 0.95
RMS_EPS = 1.1920929e-07

SS1 = 256
SS2 = 128
NIN = 8


def _fused_kernel(x_hbm, w_ref, nw_ref, bt_ref, as_ref, rw_ref, out_hbm,
                  xfirst, ring, sem_in1, sem_in2, sem_out1, sem_out2):
    B = 4
    D = D_MODEL
    HALF = 8 * SS1

    def in1_copy(j):
        return pltpu.make_async_copy(
            x_hbm.at[:, pl.ds(j * SS1, SS1), :],
            xfirst.at[:, pl.ds(j * SS1, SS1), :],
            sem_in1.at[j])

    def in2_copy(k):
        return pltpu.make_async_copy(
            x_hbm.at[:, pl.ds(HALF + k * SS2, SS2), :],
            ring.at[k % NIN],
            sem_in2.at[k % NIN])

    def out1_copy(j):
        return pltpu.make_async_copy(
            xfirst.at[:, pl.ds(j * SS1, SS1), :],
            out_hbm.at[:, pl.ds(j * SS1, SS1), :],
            sem_out1.at[j])

    def out2_copy(k):
        return pltpu.make_async_copy(
            ring.at[k % NIN],
            out_hbm.at[:, pl.ds(HALF + k * SS2, SS2), :],
            sem_out2.at[k % NIN])

    for j in range(8):
        in1_copy(j).start()
    for k in range(NIN):
        in2_copy(k).start()
    for j in range(8):
        in1_copy(j).wait()

    leaves = jnp.sum(
        xfirst[...].reshape(B, 8, SS1, D), axis=2) * (1.0 / SS1)
    states = [None] * NUM_NODES
    for n in range(7, 15):
        states[n] = leaves[:, n - 7, :]
    w = w_ref[...]
    for level in range(2, -1, -1):
        lo = (1 << level) - 1
        hi = (1 << (level + 1)) - 1
        nodes = list(range(lo, hi))
        nlev = len(nodes)
        l_st = jnp.stack([states[2 * n + 1] for n in nodes], axis=1)
        r_st = jnp.stack([states[2 * n + 2] for n in nodes], axis=1)
        fused = jnp.concatenate([l_st, r_st], axis=-1)
        fused2 = fused.reshape(B * nlev, 2 * D)
        proj = jax.lax.dot_general(
            fused2, w, (((1,), (1,)), ((), ())),
            preferred_element_type=jnp.float32,
        ).reshape(B, nlev, D)
        bt = bt_ref[level, :]
        asw = as_ref[level, :]
        v = jnp.zeros((B, D), jnp.float32)
        a = jnp.zeros((B, D), jnp.float32)
        for t in range(nlev):
            v = TAU_MEM * v + proj[:, t, :]
            thresh = bt + asw * a
            s = (v - thresh > 0).astype(jnp.float32)
            v = v * (1.0 - s)
            a = TAU_ADAPT * a + s
            states[nodes[t]] = proj[:, t, :] * s
    nw = nw_ref[...]
    mx = jnp.max(nw, axis=0, keepdims=True)
    e = jnp.exp(nw - mx)
    wts = e / jnp.sum(e, axis=0, keepdims=True)
    mix = jnp.zeros((B, D), jnp.float32)
    for n in range(NUM_NODES):
        mix = mix + wts[n, :][None, :] * states[n]
    rw = rw_ref[...]

    def rms(y):
        ms = jnp.mean(y * y, axis=-1, keepdims=True)
        return y * jax.lax.rsqrt(ms + RMS_EPS) * rw[None, None, :]

    for j in range(8):
        xblk = xfirst[:, pl.ds(j * SS1, SS1), :]
        xfirst[:, pl.ds(j * SS1, SS1), :] = rms(xblk + mix[:, None, :])
        out1_copy(j).start()

    for k in range(16):
        in2_copy(k).wait()
        blk = ring[k % NIN]
        ring[k % NIN] = rms(blk + mix[:, None, :])
        out2_copy(k).start()
        if k >= 1 and (k - 1) + NIN < 16:
            out2_copy(k - 1).wait()
            in2_copy(k - 1 + NIN).start()
    for k in range(NIN, 16):
        out2_copy(k).wait()
    for j in range(8):
        out1_copy(j).wait()


def kernel(x, W_proj, node_weights, rms_weight, base_thresh, adapt_strength):
    B, T, D = x.shape
    assert T == 4096 and D == D_MODEL, "kernel assumes T=4096, D=1024"

    out = pl.pallas_call(
        _fused_kernel,
        in_specs=[
            pl.BlockSpec(memory_space=pl.ANY),
            pl.BlockSpec(memory_space=pltpu.VMEM),
            pl.BlockSpec(memory_space=pltpu.VMEM),
            pl.BlockSpec(memory_space=pltpu.VMEM),
            pl.BlockSpec(memory_space=pltpu.VMEM),
            pl.BlockSpec(memory_space=pltpu.VMEM),
        ],
        out_specs=pl.BlockSpec(memory_space=pl.ANY),
        out_shape=jax.ShapeDtypeStruct((B, T, D), jnp.float32),
        scratch_shapes=[
            pltpu.VMEM((B, 8 * SS1, D), jnp.float32),
            pltpu.VMEM((NIN, B, SS2, D), jnp.float32),
            pltpu.SemaphoreType.DMA((8,)),
            pltpu.SemaphoreType.DMA((NIN,)),
            pltpu.SemaphoreType.DMA((8,)),
            pltpu.SemaphoreType.DMA((NIN,)),
        ],
        compiler_params=pltpu.CompilerParams(vmem_limit_bytes=62 * 1024 * 1024),
    )(x, W_proj, node_weights, base_thresh, adapt_strength, rms_weight)
    return out
